# baseline (speedup 1.0000x reference)
import functools

import jax
import jax.numpy as jnp
import numpy as np
from jax import lax
from jax.experimental import pallas as pl
from jax.experimental.pallas import tpu as pltpu
from jax.experimental.pallas import tpu_sc as plsc

D_MODEL = 64
SEQ_LEN = 200
BATCH = 1024
N_STACK = 2

NUM_CORES = 2
NUM_SUBCORES = 16

LANES = 16
BR = 128
BT = BATCH // BR
LR = 8
LT = SEQ_LEN // LR
DT = D_MODEL // 8
LQ = 4
L_PER_PHASE = SEQ_LEN // LQ

NBUF = 5
SG = 2 * NBUF
N_SG = L_PER_PHASE // SG


def _positional_encoding(depth: int, length: int) -> np.ndarray:
    half = depth / 2
    positions = np.arange(length)[:, np.newaxis]
    depths = np.arange(half)[np.newaxis, :] / half
    angle_rates = 1 / 10000 ** depths
    angle_rads = positions * angle_rates
    return np.concatenate(
        [np.sin(angle_rads), np.cos(angle_rads)], axis=-1
    ).astype(np.float32)


_POS_FLAT = _positional_encoding(D_MODEL, SEQ_LEN).reshape(-1)

_OUT5 = jax.ShapeDtypeStruct((SEQ_LEN, DT, BT, 8, BR), jnp.float32)


def _make_sc_call():
    mesh = plsc.VectorSubcoreMesh(core_axis_name="c", subcore_axis_name="s")

    scratch = [pltpu.VMEM((SEQ_LEN * D_MODEL,), jnp.float32)]
    scratch += [pltpu.VMEM((BR,), jnp.int32)] * SG
    scratch += [pltpu.VMEM((BR, D_MODEL), jnp.float32)] * NBUF
    scratch += [pltpu.VMEM((DT, 8, BR), jnp.float32)] * NBUF
    scratch += [pltpu.SemaphoreType.DMA] * (SG + 2 * NBUF)

    @functools.partial(
        pl.kernel,
        mesh=mesh,
        compiler_params=pltpu.CompilerParams(
            use_tc_tiling_on_sc=False, needs_layout_passes=False
        ),
        out_type=[_OUT5, _OUT5],
        scratch_types=scratch,
    )
    def sc_call(idx_hbm, table_hbm, pos_hbm, out0_hbm, out1_hbm, pos_v, *bufs):
        ibuf = bufs[:SG]
        gbuf = bufs[SG : SG + NBUF]
        obuf = bufs[SG + NBUF : SG + 2 * NBUF]
        isem = bufs[SG + 2 * NBUF : 2 * SG + 2 * NBUF]
        gsem = bufs[2 * SG + 2 * NBUF : 2 * SG + 3 * NBUF]
        ssem = bufs[2 * SG + 3 * NBUF :]

        wid = lax.axis_index("s") * NUM_CORES + lax.axis_index("c")
        bt = lax.rem(wid, BT)
        l0 = lax.div(wid, BT) * L_PER_PHASE

        pltpu.sync_copy(pos_hbm, pos_v)

        iota = lax.broadcasted_iota(jnp.int32, (LANES,), 0)
        ridx = [iota + 16 * grp for grp in range(BR // LANES)]

        def run_phase(t, out_hbm):
            def start_idx(l, j):
                lt = lax.div(l, LR)
                lr = lax.rem(l, LR)
                pltpu.async_copy(idx_hbm.at[t, lt, bt, lr], ibuf[j], isem[j])

            def start_gather(l_unused, j, b):
                pltpu.async_copy(table_hbm.at[ibuf[j]], gbuf[b], gsem[b])

            def wait(sem, dst):
                pltpu.make_async_copy(gbuf[0], dst, sem).wait()

            for j in range(SG):
                start_idx(l0 + j, j)
            for j in range(NBUF):
                pltpu.make_async_copy(idx_hbm.at[t, 0, 0, 0], ibuf[j], isem[j]).wait()
                start_gather(l0 + j, j, j)

            @pl.loop(0, N_SG)
            def _sg(sg):
                for j in range(SG):
                    b = j % NBUF
                    i = sg * SG + j
                    l = l0 + i
                    pltpu.make_async_copy(
                        table_hbm.at[ibuf[0]], gbuf[b], gsem[b]
                    ).wait()
                    if j < NBUF:

                        @pl.when(sg > 0)
                        def _():
                            pltpu.make_async_copy(
                                obuf[b], out_hbm.at[0, :, 0], ssem[b]
                            ).wait()

                    else:
                        pltpu.make_async_copy(
                            obuf[b], out_hbm.at[0, :, 0], ssem[b]
                        ).wait()

                    pbase = l * D_MODEL

                    @pl.loop(0, DT)
                    def _dt(dt):
                        for dr in range(8):
                            d = dt * 8 + dr
                            pos_d = plsc.load_gather(
                                pos_v, [jnp.full((LANES,), pbase + d, jnp.int32)]
                            )
                            cvec = jnp.full((LANES,), d, jnp.int32)
                            for grp in range(BR // LANES):
                                val = plsc.load_gather(
                                    gbuf[b], [ridx[grp], cvec]
                                )
                                obuf[b].at[dt, dr, pl.ds(grp * LANES, LANES)][
                                    ...
                                ] = val * 8.0 + pos_d

                    pltpu.async_copy(obuf[b], out_hbm.at[l, :, bt], ssem[b])

                    def refill():
                        pltpu.make_async_copy(
                            idx_hbm.at[t, 0, 0, 0],
                            ibuf[(j + NBUF) % SG],
                            isem[(j + NBUF) % SG],
                        ).wait()
                        start_gather(l + NBUF, (j + NBUF) % SG, b)

                    def restock_idx():
                        start_idx(l + SG, j)

                    if j < NBUF:
                        refill()

                        @pl.when(sg < N_SG - 1)
                        def _():
                            restock_idx()

                    else:

                        @pl.when(sg < N_SG - 1)
                        def _():
                            refill()
                            restock_idx()

            for b in range(NBUF):
                pltpu.make_async_copy(obuf[b], out_hbm.at[0, :, 0], ssem[b]).wait()

        run_phase(0, out0_hbm)
        run_phase(1, out1_hbm)

    return sc_call


def kernel(inputs, table):
    n_stack, batch, seq = inputs.shape
    if inputs.dtype != jnp.int32:
        inputs = inputs.astype(jnp.int32)
    idx5 = inputs.reshape(n_stack, BT, BR, LT, LR).transpose(0, 3, 1, 4, 2)
    pos = jnp.asarray(_POS_FLAT)
    out0, out1 = _make_sc_call()(idx5, table, pos)

    def _fix(o):
        return o.transpose(2, 4, 0, 1, 3).reshape(batch, seq, D_MODEL)

    return (_fix(out0), _fix(out1))


# speedup vs baseline: 1.4114x; 1.4114x over previous
import functools

import jax
import jax.numpy as jnp
import numpy as np
from jax import lax
from jax.experimental import pallas as pl
from jax.experimental.pallas import tpu as pltpu
from jax.experimental.pallas import tpu_sc as plsc

D_MODEL = 64
SEQ_LEN = 200
BATCH = 1024
N_STACK = 2

NUM_CORES = 2
NUM_SUBCORES = 16

LANES = 16
BR = 128
BT = BATCH // BR
LR = 8
LT = SEQ_LEN // LR
DT = D_MODEL // 8
LQ = 4
L_PER_PHASE = SEQ_LEN // LQ

NBUF = 5
SG = 2 * NBUF
N_SG = L_PER_PHASE // SG


def _positional_encoding(depth: int, length: int) -> np.ndarray:
    half = depth / 2
    positions = np.arange(length)[:, np.newaxis]
    depths = np.arange(half)[np.newaxis, :] / half
    angle_rates = 1 / 10000 ** depths
    angle_rads = positions * angle_rates
    return np.concatenate(
        [np.sin(angle_rads), np.cos(angle_rads)], axis=-1
    ).astype(np.float32)


_POS_FLAT = _positional_encoding(D_MODEL, SEQ_LEN).reshape(-1)

_OUT5 = jax.ShapeDtypeStruct((SEQ_LEN, DT, BT, 8, BR), jnp.float32)


def _make_sc_call():
    mesh = plsc.VectorSubcoreMesh(core_axis_name="c", subcore_axis_name="s")

    scratch = [pltpu.VMEM((SEQ_LEN * D_MODEL,), jnp.float32)]
    scratch += [pltpu.VMEM((BR,), jnp.int32)] * SG
    scratch += [pltpu.VMEM((BR, D_MODEL), jnp.float32)] * NBUF
    scratch += [pltpu.VMEM((DT, 8, BR), jnp.float32)] * NBUF
    scratch += [pltpu.SemaphoreType.DMA] * (SG + 2 * NBUF)

    @functools.partial(
        pl.kernel,
        mesh=mesh,
        compiler_params=pltpu.CompilerParams(
            use_tc_tiling_on_sc=False, needs_layout_passes=False
        ),
        out_type=[_OUT5, _OUT5],
        scratch_types=scratch,
    )
    def sc_call(idx_hbm, table_hbm, pos_hbm, out0_hbm, out1_hbm, pos_v, *bufs):
        ibuf = bufs[:SG]
        gbuf = bufs[SG : SG + NBUF]
        obuf = bufs[SG + NBUF : SG + 2 * NBUF]
        isem = bufs[SG + 2 * NBUF : 2 * SG + 2 * NBUF]
        gsem = bufs[2 * SG + 2 * NBUF : 2 * SG + 3 * NBUF]
        ssem = bufs[2 * SG + 3 * NBUF :]

        wid = lax.axis_index("s") * NUM_CORES + lax.axis_index("c")
        bt = lax.rem(wid, BT)
        l0 = lax.div(wid, BT) * L_PER_PHASE

        pltpu.sync_copy(pos_hbm, pos_v)

        iota = lax.broadcasted_iota(jnp.int32, (LANES,), 0)
        ridx = [iota + 16 * grp for grp in range(BR // LANES)]

        def run_phase(t, out_hbm):
            def start_idx(l, j):
                lt = lax.div(l, LR)
                lr = lax.rem(l, LR)
                pltpu.async_copy(idx_hbm.at[t, lt, bt, lr], ibuf[j], isem[j])

            def start_gather(l_unused, j, b):
                pltpu.async_copy(table_hbm.at[ibuf[j]], gbuf[b], gsem[b])

            def wait(sem, dst):
                pltpu.make_async_copy(gbuf[0], dst, sem).wait()

            for j in range(SG):
                start_idx(l0 + j, j)
            for j in range(NBUF):
                pltpu.make_async_copy(idx_hbm.at[t, 0, 0, 0], ibuf[j], isem[j]).wait()
                start_gather(l0 + j, j, j)

            @pl.loop(0, N_SG)
            def _sg(sg):
                for j in range(SG):
                    b = j % NBUF
                    i = sg * SG + j
                    l = l0 + i
                    pltpu.make_async_copy(
                        table_hbm.at[ibuf[0]], gbuf[b], gsem[b]
                    ).wait()
                    if j < NBUF:

                        @pl.when(sg > 0)
                        def _():
                            pltpu.make_async_copy(
                                obuf[b], out_hbm.at[0, :, 0], ssem[b]
                            ).wait()

                    else:
                        pltpu.make_async_copy(
                            obuf[b], out_hbm.at[0, :, 0], ssem[b]
                        ).wait()

                    pbase = l * D_MODEL

                    @plsc.parallel_loop(0, D_MODEL)
                    def _d(d):
                        dt = lax.div(d, 8)
                        dr = lax.rem(d, 8)
                        pos_d = plsc.load_gather(
                            pos_v, [jnp.full((LANES,), pbase + d, jnp.int32)]
                        )
                        cvec = jnp.full((LANES,), d, jnp.int32)
                        for grp in range(BR // LANES):
                            val = plsc.load_gather(gbuf[b], [ridx[grp], cvec])
                            obuf[b].at[dt, dr, pl.ds(grp * LANES, LANES)][
                                ...
                            ] = val * 8.0 + pos_d

                    pltpu.async_copy(obuf[b], out_hbm.at[l, :, bt], ssem[b])

                    def refill():
                        pltpu.make_async_copy(
                            idx_hbm.at[t, 0, 0, 0],
                            ibuf[(j + NBUF) % SG],
                            isem[(j + NBUF) % SG],
                        ).wait()
                        start_gather(l + NBUF, (j + NBUF) % SG, b)

                    def restock_idx():
                        start_idx(l + SG, j)

                    if j < NBUF:
                        refill()

                        @pl.when(sg < N_SG - 1)
                        def _():
                            restock_idx()

                    else:

                        @pl.when(sg < N_SG - 1)
                        def _():
                            refill()
                            restock_idx()

            for b in range(NBUF):
                pltpu.make_async_copy(obuf[b], out_hbm.at[0, :, 0], ssem[b]).wait()

        run_phase(0, out0_hbm)
        run_phase(1, out1_hbm)

    return sc_call


def kernel(inputs, table):
    n_stack, batch, seq = inputs.shape
    if inputs.dtype != jnp.int32:
        inputs = inputs.astype(jnp.int32)
    idx5 = inputs.reshape(n_stack, BT, BR, LT, LR).transpose(0, 3, 1, 4, 2)
    pos = jnp.asarray(_POS_FLAT)
    out0, out1 = _make_sc_call()(idx5, table, pos)

    def _fix(o):
        return o.transpose(2, 4, 0, 1, 3).reshape(batch, seq, D_MODEL)

    return (_fix(out0), _fix(out1))


# speedup vs baseline: 1.4118x; 1.0003x over previous
import functools

import jax
import jax.numpy as jnp
import numpy as np
from jax import lax
from jax.experimental import pallas as pl
from jax.experimental.pallas import tpu as pltpu
from jax.experimental.pallas import tpu_sc as plsc

D_MODEL = 64
SEQ_LEN = 200
BATCH = 1024
N_STACK = 2

NUM_CORES = 2
NUM_SUBCORES = 16

LANES = 16
BR = 128
BT = BATCH // BR
LR = 8
LT = SEQ_LEN // LR
DT = D_MODEL // 8
LQ = 4
L_PER_PHASE = SEQ_LEN // LQ

NBUF = 5
SG = 2 * NBUF
N_SG = L_PER_PHASE // SG


def _positional_encoding(depth: int, length: int) -> np.ndarray:
    half = depth / 2
    positions = np.arange(length)[:, np.newaxis]
    depths = np.arange(half)[np.newaxis, :] / half
    angle_rates = 1 / 10000 ** depths
    angle_rads = positions * angle_rates
    return np.concatenate(
        [np.sin(angle_rads), np.cos(angle_rads)], axis=-1
    ).astype(np.float32)


_POS_FLAT = _positional_encoding(D_MODEL, SEQ_LEN).reshape(-1)

_OUT5 = jax.ShapeDtypeStruct((SEQ_LEN, DT, BT, 8, BR), jnp.float32)


def _make_sc_call():
    mesh = plsc.VectorSubcoreMesh(core_axis_name="c", subcore_axis_name="s")

    scratch = [pltpu.VMEM((SEQ_LEN * D_MODEL,), jnp.float32)]
    scratch += [pltpu.VMEM((BR,), jnp.int32)] * SG
    scratch += [pltpu.VMEM((BR, D_MODEL), jnp.float32)] * NBUF
    scratch += [pltpu.VMEM((DT, 8, BR), jnp.float32)] * NBUF
    scratch += [pltpu.SemaphoreType.DMA] * (SG + 2 * NBUF)

    @functools.partial(
        pl.kernel,
        mesh=mesh,
        compiler_params=pltpu.CompilerParams(
            use_tc_tiling_on_sc=False, needs_layout_passes=False
        ),
        out_type=[_OUT5, _OUT5],
        scratch_types=scratch,
    )
    def sc_call(idx_hbm, table_hbm, pos_hbm, out0_hbm, out1_hbm, pos_v, *bufs):
        ibuf = bufs[:SG]
        gbuf = bufs[SG : SG + NBUF]
        obuf = bufs[SG + NBUF : SG + 2 * NBUF]
        isem = bufs[SG + 2 * NBUF : 2 * SG + 2 * NBUF]
        gsem = bufs[2 * SG + 2 * NBUF : 2 * SG + 3 * NBUF]
        ssem = bufs[2 * SG + 3 * NBUF :]

        wid = lax.axis_index("s") * NUM_CORES + lax.axis_index("c")
        bt = lax.rem(wid, BT)
        l0 = lax.div(wid, BT) * L_PER_PHASE

        pltpu.sync_copy(pos_hbm, pos_v)

        iota = lax.broadcasted_iota(jnp.int32, (LANES,), 0)
        ridx = [iota + 16 * grp for grp in range(BR // LANES)]

        def run_phase(t, out_hbm):
            def start_idx(l, j):
                lt = lax.div(l, LR)
                lr = lax.rem(l, LR)
                pltpu.async_copy(idx_hbm.at[t, lt, bt, lr], ibuf[j], isem[j])

            def start_gather(l_unused, j, b):
                pltpu.async_copy(table_hbm.at[ibuf[j]], gbuf[b], gsem[b])

            def wait(sem, dst):
                pltpu.make_async_copy(gbuf[0], dst, sem).wait()

            for j in range(SG):
                start_idx(l0 + j, j)
            for j in range(NBUF):
                pltpu.make_async_copy(idx_hbm.at[t, 0, 0, 0], ibuf[j], isem[j]).wait()
                start_gather(l0 + j, j, j)

            @pl.loop(0, N_SG)
            def _sg(sg):
                for j in range(SG):
                    b = j % NBUF
                    i = sg * SG + j
                    l = l0 + i
                    pltpu.make_async_copy(
                        table_hbm.at[ibuf[0]], gbuf[b], gsem[b]
                    ).wait()
                    if j < NBUF:

                        @pl.when(sg > 0)
                        def _():
                            pltpu.make_async_copy(
                                obuf[b], out_hbm.at[0, :, 0], ssem[b]
                            ).wait()

                    else:
                        pltpu.make_async_copy(
                            obuf[b], out_hbm.at[0, :, 0], ssem[b]
                        ).wait()

                    pbase = l * D_MODEL

                    @plsc.parallel_loop(0, D_MODEL)
                    def _d(d):
                        dt = lax.div(d, 8)
                        dr = lax.rem(d, 8)
                        pos_d = plsc.load_gather(
                            pos_v, [jnp.full((LANES,), pbase + d, jnp.int32)]
                        )
                        cvec = jnp.full((LANES,), d, jnp.int32)
                        for grp in range(BR // LANES):
                            val = plsc.load_gather(gbuf[b], [ridx[grp], cvec])
                            obuf[b].at[dt, dr, pl.ds(grp * LANES, LANES)][
                                ...
                            ] = val * 8.0 + pos_d

                    for dt_s in range(DT):
                        pltpu.async_copy(
                            obuf[b].at[dt_s], out_hbm.at[l, dt_s, bt], ssem[b]
                        )

                    def refill():
                        pltpu.make_async_copy(
                            idx_hbm.at[t, 0, 0, 0],
                            ibuf[(j + NBUF) % SG],
                            isem[(j + NBUF) % SG],
                        ).wait()
                        start_gather(l + NBUF, (j + NBUF) % SG, b)

                    def restock_idx():
                        start_idx(l + SG, j)

                    if j < NBUF:
                        refill()

                        @pl.when(sg < N_SG - 1)
                        def _():
                            restock_idx()

                    else:

                        @pl.when(sg < N_SG - 1)
                        def _():
                            refill()
                            restock_idx()

            for b in range(NBUF):
                pltpu.make_async_copy(obuf[b], out_hbm.at[0, :, 0], ssem[b]).wait()

        run_phase(0, out0_hbm)
        run_phase(1, out1_hbm)

    return sc_call


def kernel(inputs, table):
    n_stack, batch, seq = inputs.shape
    if inputs.dtype != jnp.int32:
        inputs = inputs.astype(jnp.int32)
    idx5 = inputs.reshape(n_stack, BT, BR, LT, LR).transpose(0, 3, 1, 4, 2)
    pos = jnp.asarray(_POS_FLAT)
    out0, out1 = _make_sc_call()(idx5, table, pos)

    def _fix(o):
        return o.transpose(2, 4, 0, 1, 3).reshape(batch, seq, D_MODEL)

    return (_fix(out0), _fix(out1))


# speedup vs baseline: 1.4176x; 1.0041x over previous
import functools

import jax
import jax.numpy as jnp
import numpy as np
from jax import lax
from jax.experimental import pallas as pl
from jax.experimental.pallas import tpu as pltpu
from jax.experimental.pallas import tpu_sc as plsc

D_MODEL = 64
SEQ_LEN = 200
BATCH = 1024
N_STACK = 2

NUM_CORES = 2
NUM_SUBCORES = 16

LANES = 16
BR = 128
BT = BATCH // BR
LR = 8
LT = SEQ_LEN // LR
DT = D_MODEL // 8
LQ = 4
L_PER_PHASE = SEQ_LEN // LQ

NBUF = 5
SG = 2 * NBUF
N_SG = L_PER_PHASE // SG


def _positional_encoding(depth: int, length: int) -> np.ndarray:
    half = depth / 2
    positions = np.arange(length)[:, np.newaxis]
    depths = np.arange(half)[np.newaxis, :] / half
    angle_rates = 1 / 10000 ** depths
    angle_rads = positions * angle_rates
    return np.concatenate(
        [np.sin(angle_rads), np.cos(angle_rads)], axis=-1
    ).astype(np.float32)


_POS_FLAT = _positional_encoding(D_MODEL, SEQ_LEN).reshape(-1)

_OUT5 = jax.ShapeDtypeStruct((SEQ_LEN, DT, BT, 8, BR), jnp.float32)


def _make_sc_call():
    mesh = plsc.VectorSubcoreMesh(core_axis_name="c", subcore_axis_name="s")

    scratch = [pltpu.VMEM((SEQ_LEN * D_MODEL,), jnp.float32)]
    scratch += [pltpu.VMEM((BR,), jnp.int32)] * SG
    scratch += [pltpu.VMEM((BR, D_MODEL), jnp.float32)] * NBUF
    scratch += [pltpu.VMEM((DT, 8, BR), jnp.float32)] * NBUF
    scratch += [pltpu.SemaphoreType.DMA] * (SG + 2 * NBUF)

    @functools.partial(
        pl.kernel,
        mesh=mesh,
        compiler_params=pltpu.CompilerParams(
            use_tc_tiling_on_sc=False, needs_layout_passes=False
        ),
        out_type=[_OUT5, _OUT5],
        scratch_types=scratch,
    )
    def sc_call(idx_hbm, table_hbm, pos_hbm, out0_hbm, out1_hbm, pos_v, *bufs):
        ibuf = bufs[:SG]
        gbuf = bufs[SG : SG + NBUF]
        obuf = bufs[SG + NBUF : SG + 2 * NBUF]
        isem = bufs[SG + 2 * NBUF : 2 * SG + 2 * NBUF]
        gsem = bufs[2 * SG + 2 * NBUF : 2 * SG + 3 * NBUF]
        ssem = bufs[2 * SG + 3 * NBUF :]

        wid = lax.axis_index("s") * NUM_CORES + lax.axis_index("c")
        bt = lax.rem(wid, BT)
        l0 = lax.div(wid, BT) * L_PER_PHASE

        pltpu.sync_copy(pos_hbm, pos_v)

        iota = lax.broadcasted_iota(jnp.int32, (LANES,), 0)
        ridx = [iota + 16 * grp for grp in range(BR // LANES)]

        def run_phase(t, out_hbm):
            def start_idx(l, j):
                lt = lax.div(l, LR)
                lr = lax.rem(l, LR)
                pltpu.async_copy(idx_hbm.at[t, lt, bt, lr], ibuf[j], isem[j])

            def start_gather(l_unused, j, b):
                pltpu.async_copy(table_hbm.at[ibuf[j]], gbuf[b], gsem[b])

            def wait(sem, dst):
                pltpu.make_async_copy(gbuf[0], dst, sem).wait()

            for j in range(SG):
                start_idx(l0 + j, j)
            for j in range(NBUF):
                pltpu.make_async_copy(idx_hbm.at[t, 0, 0, 0], ibuf[j], isem[j]).wait()
                start_gather(l0 + j, j, j)

            @pl.loop(0, N_SG)
            def _sg(sg):
                for j in range(SG):
                    b = j % NBUF
                    i = sg * SG + j
                    l = l0 + i
                    pltpu.make_async_copy(
                        table_hbm.at[ibuf[0]], gbuf[b], gsem[b]
                    ).wait()
                    if j < NBUF:

                        @pl.when(sg > 0)
                        def _():
                            pltpu.make_async_copy(
                                obuf[b], out_hbm.at[0, :, 0], ssem[b]
                            ).wait()

                    else:
                        pltpu.make_async_copy(
                            obuf[b], out_hbm.at[0, :, 0], ssem[b]
                        ).wait()

                    pbase = l * D_MODEL

                    @plsc.parallel_loop(0, D_MODEL)
                    def _d(d):
                        dt = lax.div(d, 8)
                        dr = lax.rem(d, 8)
                        pos_d = plsc.load_gather(
                            pos_v, [jnp.full((LANES,), pbase + d, jnp.int32)]
                        )
                        cvec = jnp.full((LANES,), d, jnp.int32)
                        for grp in range(BR // LANES):
                            val = plsc.load_gather(gbuf[b], [ridx[grp], cvec])
                            obuf[b].at[dt, dr, pl.ds(grp * LANES, LANES)][
                                ...
                            ] = val * 8.0 + pos_d

                    pltpu.async_copy(obuf[b], out_hbm.at[l, :, bt], ssem[b])

                    def refill():
                        pltpu.make_async_copy(
                            idx_hbm.at[t, 0, 0, 0],
                            ibuf[(j + NBUF) % SG],
                            isem[(j + NBUF) % SG],
                        ).wait()
                        start_gather(l + NBUF, (j + NBUF) % SG, b)

                    def restock_idx():
                        start_idx(l + SG, j)

                    if j < NBUF:
                        refill()

                        @pl.when(sg < N_SG - 1)
                        def _():
                            restock_idx()

                    else:

                        @pl.when(sg < N_SG - 1)
                        def _():
                            refill()
                            restock_idx()

            for b in range(NBUF):
                pltpu.make_async_copy(obuf[b], out_hbm.at[0, :, 0], ssem[b]).wait()

        run_phase(0, out0_hbm)
        run_phase(1, out1_hbm)

    return sc_call


def kernel(inputs, table):
    n_stack, batch, seq = inputs.shape
    if inputs.dtype != jnp.int32:
        inputs = inputs.astype(jnp.int32)
    idx5 = inputs.reshape(n_stack, BT, BR, LT, LR).transpose(0, 3, 1, 4, 2)
    pos = jnp.asarray(_POS_FLAT)
    out0, out1 = _make_sc_call()(idx5, table, pos)

    def _fix(o):
        return o.transpose(2, 4, 0, 1, 3).reshape(batch, seq, D_MODEL)

    return (_fix(out0), _fix(out1))
